# Initial kernel scaffold; baseline (speedup 1.0000x reference)
import jax
import jax.numpy as jnp
from jax.experimental import pallas as pl


def kernel(xyz, points, new_xyz, new_points, affine_alpha, affine_beta):
    raise NotImplementedError("write your pallas kernel here")


# speedup vs baseline: 4.1376x; 4.1376x over previous
import functools

import jax
import jax.numpy as jnp
from jax import lax
from jax.experimental import pallas as pl
from jax.experimental.pallas import tpu as pltpu
from jax.experimental.pallas import tpu_sc as plsc

B, N, S, K, D = 8, 2048, 512, 32, 256
NC, NS, L = 2, 16, 16
NW = NC * NS
BS = B * S
PAIRS_PER_W = BS // NW
PAIRS_PER_BATCH = S
DI = D // L


def _topk_body(nxyz_ref, xyzt_ref, idx_ref):
    b = pl.program_id(0)
    sx = nxyz_ref[0]
    dx = xyzt_ref[0]
    s0 = sx[:, 0:1]
    s1 = sx[:, 1:2]
    s2 = sx[:, 2:3]
    d0 = dx[0:1, :]
    d1 = dx[1:2, :]
    d2 = dx[2:3, :]

    def _r(x):
        return x.astype(jnp.bfloat16).astype(jnp.float32)

    dot = (_r(s0) * _r(d0) + _r(s1) * _r(d1)) + _r(s2) * _r(d2)
    sn = (s0 * s0 + s1 * s1) + s2 * s2
    dn = (d0 * d0 + d1 * d1) + d2 * d2
    dist = (-2.0 * dot + sn) + dn

    lane = lax.broadcasted_iota(jnp.int32, (S, N), 1)
    kcol = lax.broadcasted_iota(jnp.int32, (S, K), 1)
    idxs = jnp.zeros((S, K), jnp.int32)
    inf = jnp.float32(jnp.inf)
    for k in range(K):
        m = jnp.min(dist, axis=1, keepdims=True)
        eq = dist == m
        cand = jnp.where(eq, lane, N)
        a = jnp.min(cand, axis=1, keepdims=True)
        idxs = jnp.where(kcol == k, a, idxs)
        dist = jnp.where(lane == a, inf, dist)
    idx_ref[0] = idxs + b * N


def _topk_indices(new_xyz, xyz):
    xyzt = jnp.transpose(xyz, (0, 2, 1))
    return pl.pallas_call(
        _topk_body,
        grid=(B,),
        in_specs=[
            pl.BlockSpec((1, S, 3), lambda b: (b, 0, 0)),
            pl.BlockSpec((1, 3, N), lambda b: (b, 0, 0)),
        ],
        out_specs=pl.BlockSpec((1, S, K), lambda b: (b, 0, 0)),
        out_shape=jax.ShapeDtypeStruct((B, S, K), jnp.int32),
    )(new_xyz, xyzt)


@functools.cache
def _mesh():
    return plsc.VectorSubcoreMesh(core_axis_name="c", subcore_axis_name="s")


def _sc_stats_body(points_hbm, np_hbm, idx_hbm, out_hbm,
                   idxv, rowsv, npv, outv, sem):
    wid = lax.axis_index("s") * NC + lax.axis_index("c")
    base = wid * PAIRS_PER_W

    zero = jnp.zeros((L,), jnp.float32)
    acc_init = tuple([zero] * DI) + tuple([zero] * DI)

    def pair_body(j, accs):
        p = base + j
        pltpu.sync_copy(idx_hbm.at[p], idxv)
        pltpu.async_copy(points_hbm.at[idxv], rowsv, sem).wait()
        pltpu.sync_copy(np_hbm.at[p], npv)
        accs = list(accs)
        for i in range(DI):
            npi = npv[pl.ds(i * L, L)]
            a1 = accs[i]
            a2 = accs[DI + i]
            for k in range(K):
                r = rowsv[k, pl.ds(i * L, L)] - npi
                a1 = a1 + r
                a2 = a2 + r * r
            accs[i] = a1
            accs[DI + i] = a2
        return tuple(accs)

    accs = lax.fori_loop(0, PAIRS_PER_W, pair_body, acc_init)
    a1 = accs[0]
    a2 = accs[DI]
    for i in range(1, DI):
        a1 = a1 + accs[i]
        a2 = a2 + accs[DI + i]
    outv[pl.ds(0, L)] = a1
    outv[pl.ds(L, L)] = a2
    pltpu.sync_copy(outv, out_hbm.at[wid])


@functools.cache
def _sc_stats():
    return pl.kernel(
        _sc_stats_body,
        out_type=jax.ShapeDtypeStruct((NW, 2 * L), jnp.float32),
        mesh=_mesh(),
        scratch_types=[
            pltpu.VMEM((K,), jnp.int32),
            pltpu.VMEM((K, D), jnp.float32),
            pltpu.VMEM((D,), jnp.float32),
            pltpu.VMEM((2 * L,), jnp.float32),
            pltpu.SemaphoreType.DMA,
        ],
    )


def _sc_out_body(points_hbm, np_hbm, idx_hbm, sa_hbm, beta_hbm, out_hbm,
                 idxv, rowsv, npv, sav, betav, outt, sem):
    wid = lax.axis_index("s") * NC + lax.axis_index("c")
    base = wid * PAIRS_PER_W
    bidx = base // PAIRS_PER_BATCH

    pltpu.sync_copy(sa_hbm.at[bidx], sav)
    pltpu.sync_copy(beta_hbm.at[0], betav)

    def pair_body(j, carry):
        p = base + j
        pltpu.sync_copy(idx_hbm.at[p], idxv)
        pltpu.async_copy(points_hbm.at[idxv], rowsv, sem).wait()
        pltpu.sync_copy(np_hbm.at[p], npv)
        for i in range(DI):
            npi = npv[pl.ds(i * L, L)]
            sai = sav[pl.ds(i * L, L)]
            bi = betav[pl.ds(i * L, L)]
            for k in range(K):
                r = rowsv[k, pl.ds(i * L, L)]
                outt[k, pl.ds(i * L, L)] = (r - npi) * sai + bi
                outt[k, pl.ds(D + i * L, L)] = npi
        pltpu.sync_copy(outt, out_hbm.at[pl.ds(p * K, K)])
        return carry

    lax.fori_loop(0, PAIRS_PER_W, pair_body, 0)


@functools.cache
def _sc_out():
    return pl.kernel(
        _sc_out_body,
        out_type=jax.ShapeDtypeStruct((BS * K, 2 * D), jnp.float32),
        mesh=_mesh(),
        scratch_types=[
            pltpu.VMEM((K,), jnp.int32),
            pltpu.VMEM((K, D), jnp.float32),
            pltpu.VMEM((D,), jnp.float32),
            pltpu.VMEM((D,), jnp.float32),
            pltpu.VMEM((D,), jnp.float32),
            pltpu.VMEM((K, 2 * D), jnp.float32),
            pltpu.SemaphoreType.DMA,
        ],
    )


def kernel(xyz, points, new_xyz, new_points, affine_alpha, affine_beta):
    idx = _topk_indices(new_xyz, xyz)

    points_flat = points.reshape(B * N, D)
    np_flat = new_points.reshape(BS, D)
    idx_flat = idx.reshape(BS, K)

    stats = _sc_stats()(points_flat, np_flat, idx_flat)
    per_batch = stats.reshape(B, NW // B, 2, L)
    sums = jnp.sum(per_batch[:, :, 0, :], axis=(1, 2))
    sumsqs = jnp.sum(per_batch[:, :, 1, :], axis=(1, 2))
    n = jnp.float32(S * K * D)
    var = (sumsqs - sums * sums / n) / (n - 1.0)
    std = jnp.sqrt(var)
    alpha = affine_alpha.reshape(1, D)
    beta = affine_beta.reshape(1, D)
    sa = alpha / (std[:, None] + 1e-05)

    out = _sc_out()(points_flat, np_flat, idx_flat, sa, beta)
    return (new_xyz, out.reshape(B, S, K, 2 * D))


# speedup vs baseline: 6.2391x; 1.5079x over previous
import functools

import jax
import jax.numpy as jnp
from jax import lax
from jax.experimental import pallas as pl
from jax.experimental.pallas import tpu as pltpu
from jax.experimental.pallas import tpu_sc as plsc

B, N, S, K, D = 8, 2048, 512, 32, 256
NC, NS, L = 2, 16, 16
NW = NC * NS
BS = B * S
PAIRS_PER_W = BS // NW
PAIRS_PER_BATCH = S
DI = D // L


def _topk_body(nxyz_ref, xyzt_ref, idx_ref):
    b = pl.program_id(0)
    sx = nxyz_ref[0]
    dx = xyzt_ref[0]
    s0 = sx[:, 0:1]
    s1 = sx[:, 1:2]
    s2 = sx[:, 2:3]
    d0 = dx[0:1, :]
    d1 = dx[1:2, :]
    d2 = dx[2:3, :]

    def _r(x):
        return x.astype(jnp.bfloat16).astype(jnp.float32)

    dot = (_r(s0) * _r(d0) + _r(s1) * _r(d1)) + _r(s2) * _r(d2)
    sn = (s0 * s0 + s1 * s1) + s2 * s2
    dn = (d0 * d0 + d1 * d1) + d2 * d2
    dist = (-2.0 * dot + sn) + dn

    lane = lax.broadcasted_iota(jnp.int32, (S, N), 1)
    kcol = lax.broadcasted_iota(jnp.int32, (S, K), 1)
    idxs = jnp.zeros((S, K), jnp.int32)
    inf = jnp.float32(jnp.inf)
    for k in range(K):
        m = jnp.min(dist, axis=1, keepdims=True)
        eq = dist == m
        cand = jnp.where(eq, lane, N)
        a = jnp.min(cand, axis=1, keepdims=True)
        idxs = jnp.where(kcol == k, a, idxs)
        dist = jnp.where(lane == a, inf, dist)
    idx_ref[0] = idxs + b * N


def _topk_indices(new_xyz, xyz):
    xyzt = jnp.transpose(xyz, (0, 2, 1))
    return pl.pallas_call(
        _topk_body,
        grid=(B,),
        in_specs=[
            pl.BlockSpec((1, S, 3), lambda b: (b, 0, 0)),
            pl.BlockSpec((1, 3, N), lambda b: (b, 0, 0)),
        ],
        out_specs=pl.BlockSpec((1, S, K), lambda b: (b, 0, 0)),
        out_shape=jax.ShapeDtypeStruct((B, S, K), jnp.int32),
    )(new_xyz, xyzt)


@functools.cache
def _mesh():
    return plsc.VectorSubcoreMesh(core_axis_name="c", subcore_axis_name="s")


_NACC = 4


def _sc_stats_body(points_hbm, np_hbm, idx_hbm, out_hbm,
                   idxslab, npslab, rowsv, outv, gsem0, gsem1):
    wid = lax.axis_index("s") * NC + lax.axis_index("c")
    base = wid * PAIRS_PER_W

    pltpu.sync_copy(idx_hbm.at[pl.ds(base, PAIRS_PER_W)], idxslab)
    pltpu.sync_copy(np_hbm.at[pl.ds(base, PAIRS_PER_W)], npslab)

    def start_gather(j, slot, sem):
        pltpu.async_copy(points_hbm.at[idxslab.at[j]], rowsv.at[slot], sem)

    def wait_gather(j, slot, sem):
        pltpu.make_async_copy(points_hbm.at[idxslab.at[j]],
                              rowsv.at[slot], sem).wait()

    start_gather(0, 0, gsem0)
    start_gather(1, 1, gsem1)

    zero = jnp.zeros((L,), jnp.float32)
    acc_init = tuple([zero] * (2 * _NACC))

    def accum_pair(j, slot, accs):
        accs = list(accs)
        for i in range(DI):
            npi = npslab[j, pl.ds(i * L, L)]
            a1 = accs[i % _NACC]
            a2 = accs[_NACC + i % _NACC]
            for k in range(K):
                r = rowsv[slot, k, pl.ds(i * L, L)] - npi
                a1 = a1 + r
                a2 = a2 + r * r
            accs[i % _NACC] = a1
            accs[_NACC + i % _NACC] = a2
        return tuple(accs)

    def body(jj, accs):
        j0 = 2 * jj
        wait_gather(j0, 0, gsem0)

        @pl.when(jj < PAIRS_PER_W // 2 - 1)
        def _():
            start_gather(j0 + 2, 0, gsem0)

        accs = accum_pair(j0, 0, accs)
        wait_gather(j0 + 1, 1, gsem1)

        @pl.when(jj < PAIRS_PER_W // 2 - 1)
        def _():
            start_gather(j0 + 3, 1, gsem1)

        return accum_pair(j0 + 1, 1, accs)

    accs = lax.fori_loop(0, PAIRS_PER_W // 2, body, acc_init)
    a1 = accs[0]
    a2 = accs[_NACC]
    for i in range(1, _NACC):
        a1 = a1 + accs[i]
        a2 = a2 + accs[_NACC + i]
    outv[pl.ds(0, L)] = a1
    outv[pl.ds(L, L)] = a2
    pltpu.sync_copy(outv, out_hbm.at[wid])


@functools.cache
def _sc_stats():
    return pl.kernel(
        _sc_stats_body,
        out_type=jax.ShapeDtypeStruct((NW, 2 * L), jnp.float32),
        mesh=_mesh(),
        scratch_types=[
            pltpu.VMEM((PAIRS_PER_W, K), jnp.int32),
            pltpu.VMEM((PAIRS_PER_W, D), jnp.float32),
            pltpu.VMEM((2, K, D), jnp.float32),
            pltpu.VMEM((2 * L,), jnp.float32),
            pltpu.SemaphoreType.DMA,
            pltpu.SemaphoreType.DMA,
        ],
    )


def _sc_out_body(points_hbm, np_hbm, idx_hbm, sa_hbm, beta_hbm, out_hbm,
                 idxslab, npslab, rowsv, sav, betav, outt,
                 gsem0, gsem1, wsem0, wsem1):
    wid = lax.axis_index("s") * NC + lax.axis_index("c")
    base = wid * PAIRS_PER_W
    bidx = base // PAIRS_PER_BATCH

    pltpu.sync_copy(sa_hbm.at[bidx], sav)
    pltpu.sync_copy(beta_hbm.at[0], betav)
    pltpu.sync_copy(idx_hbm.at[pl.ds(base, PAIRS_PER_W)], idxslab)
    pltpu.sync_copy(np_hbm.at[pl.ds(base, PAIRS_PER_W)], npslab)

    def start_gather(j, slot, sem):
        pltpu.async_copy(points_hbm.at[idxslab.at[j]], rowsv.at[slot], sem)

    def wait_gather(j, slot, sem):
        pltpu.make_async_copy(points_hbm.at[idxslab.at[j]],
                              rowsv.at[slot], sem).wait()

    def start_write(j, slot, sem):
        pltpu.async_copy(outt.at[slot],
                         out_hbm.at[pl.ds((base + j) * K, K)], sem)

    def wait_write(j, slot, sem):
        pltpu.make_async_copy(outt.at[slot],
                              out_hbm.at[pl.ds((base + j) * K, K)], sem).wait()

    start_gather(0, 0, gsem0)
    start_gather(1, 1, gsem1)

    def compute_pair(j, slot):
        for i in range(DI):
            npi = npslab[j, pl.ds(i * L, L)]
            sai = sav[pl.ds(i * L, L)]
            bi = betav[pl.ds(i * L, L)]
            for k in range(K):
                r = rowsv[slot, k, pl.ds(i * L, L)]
                outt[slot, k, pl.ds(i * L, L)] = (r - npi) * sai + bi
                outt[slot, k, pl.ds(D + i * L, L)] = npi

    def half(jj, j, slot, gsem, wsem):
        wait_gather(j, slot, gsem)

        @pl.when(jj > 0)
        def _():
            wait_write(j - 2, slot, wsem)

        compute_pair(j, slot)
        start_write(j, slot, wsem)

        @pl.when(jj < PAIRS_PER_W // 2 - 1)
        def _():
            start_gather(j + 2, slot, gsem)

    def body(jj, carry):
        j0 = 2 * jj
        half(jj, j0, 0, gsem0, wsem0)
        half(jj, j0 + 1, 1, gsem1, wsem1)
        return carry

    lax.fori_loop(0, PAIRS_PER_W // 2, body, 0)
    wait_write(PAIRS_PER_W - 2, 0, wsem0)
    wait_write(PAIRS_PER_W - 1, 1, wsem1)


@functools.cache
def _sc_out():
    return pl.kernel(
        _sc_out_body,
        out_type=jax.ShapeDtypeStruct((BS * K, 2 * D), jnp.float32),
        mesh=_mesh(),
        scratch_types=[
            pltpu.VMEM((PAIRS_PER_W, K), jnp.int32),
            pltpu.VMEM((PAIRS_PER_W, D), jnp.float32),
            pltpu.VMEM((2, K, D), jnp.float32),
            pltpu.VMEM((D,), jnp.float32),
            pltpu.VMEM((D,), jnp.float32),
            pltpu.VMEM((2, K, 2 * D), jnp.float32),
            pltpu.SemaphoreType.DMA,
            pltpu.SemaphoreType.DMA,
            pltpu.SemaphoreType.DMA,
            pltpu.SemaphoreType.DMA,
        ],
    )


def kernel(xyz, points, new_xyz, new_points, affine_alpha, affine_beta):
    idx = _topk_indices(new_xyz, xyz)

    points_flat = points.reshape(B * N, D)
    np_flat = new_points.reshape(BS, D)
    idx_flat = idx.reshape(BS, K)

    stats = _sc_stats()(points_flat, np_flat, idx_flat)
    per_batch = stats.reshape(B, NW // B, 2, L)
    sums = jnp.sum(per_batch[:, :, 0, :], axis=(1, 2))
    sumsqs = jnp.sum(per_batch[:, :, 1, :], axis=(1, 2))
    n = jnp.float32(S * K * D)
    var = (sumsqs - sums * sums / n) / (n - 1.0)
    std = jnp.sqrt(var)
    alpha = affine_alpha.reshape(1, D)
    beta = affine_beta.reshape(1, D)
    sa = alpha / (std[:, None] + 1e-05)

    out = _sc_out()(points_flat, np_flat, idx_flat, sa, beta)
    return (new_xyz, out.reshape(B, S, K, 2 * D))
